# Initial kernel scaffold; baseline (speedup 1.0000x reference)
import functools

import jax
import jax.numpy as jnp
from jax import lax
from jax.experimental import pallas as pl
from jax.experimental.pallas import tpu as pltpu
from jax.experimental.pallas import tpu_sc as plsc

N = 10000
E = 160000
DP = 128
HP = 320
HH = HP // 2
DEGC = 300 - HH
NCORE = 2
NSUB = 16
EPT = E // NSUB
CH = 80
NCH = EPT // CH
RC = 125
NRC = N // NSUB // RC
NB = 1000
GP = 16
CPAD = 128

_dot = functools.partial(jnp.dot, precision=lax.Precision.HIGHEST,
                         preferred_element_type=jnp.float32)


def _sc_edge_pass(h_stack, src_off, dst_idx, zeros_blk):
  mesh = plsc.VectorSubcoreMesh(core_axis_name="c", subcore_axis_name="s")

  @functools.partial(
      pl.kernel,
      out_type=jax.ShapeDtypeStruct((NCORE, N, HH), jnp.float32),
      mesh=mesh,
      scratch_types=[
          pltpu.VMEM((NCH, CH), jnp.int32),
          pltpu.VMEM((NCH, CH), jnp.int32),
          pltpu.VMEM((CH, HH), jnp.float32),
          pltpu.VMEM((RC, HH), jnp.float32),
          pltpu.VMEM_SHARED((N, HH), jnp.float32),
          pltpu.SemaphoreType.DMA,
      ],
  )
  def k(h_hbm, src_hbm, dst_hbm, z_hbm, out_hbm,
        src_v, dst_v, rows_v, zb_v, agg_sh, sem):
    cid = lax.axis_index("c")
    sid = lax.axis_index("s")
    base = sid * (N // NSUB)
    pltpu.sync_copy(z_hbm, zb_v)
    for j in range(NRC):
      pltpu.sync_copy(zb_v, agg_sh.at[pl.ds(base + j * RC, RC)])
    pltpu.sync_copy(src_hbm.at[cid, sid], src_v)
    pltpu.sync_copy(dst_hbm.at[sid], dst_v)
    plsc.subcore_barrier()

    def body(kk, carry):
      pltpu.async_copy(h_hbm.at[src_v.at[kk]], rows_v, sem).wait()
      pltpu.sync_copy(rows_v, agg_sh.at[dst_v.at[kk]], add=True)
      return carry

    lax.fori_loop(0, NCH, body, 0)

    plsc.subcore_barrier()
    for j in range(NRC):
      pltpu.sync_copy(agg_sh.at[pl.ds(base + j * RC, RC)], zb_v)
      pltpu.sync_copy(zb_v, out_hbm.at[cid, pl.ds(base + j * RC, RC)])

  return k(h_stack, src_off, dst_idx, zeros_blk)


def _split_store(ref, val):
  ref[0] = val[:, :HH]
  ref[1] = val[:, HH:]


def _tc_lift(nf, Wl, bl, W2, b2):
  def body(nf_ref, Wl_ref, bl_ref, W2_ref, b2_ref, x_ref, h_ref):
    x0 = _dot(nf_ref[...], Wl_ref[...]) + bl_ref[...]
    h1 = jnp.maximum(_dot(x0, W2_ref[...]) + b2_ref[...], 0.0)
    _split_store(x_ref, x0)
    _split_store(h_ref, h1)

  return pl.pallas_call(
      body,
      grid=(N // NB,),
      in_specs=[
          pl.BlockSpec((NB, DP), lambda i: (i, 0)),
          pl.BlockSpec((DP, HP), lambda i: (0, 0)),
          pl.BlockSpec((1, HP), lambda i: (0, 0)),
          pl.BlockSpec((HP, HP), lambda i: (0, 0)),
          pl.BlockSpec((1, HP), lambda i: (0, 0)),
      ],
      out_specs=[
          pl.BlockSpec((NCORE, NB, HH), lambda i: (0, i, 0)),
          pl.BlockSpec((NCORE, NB, HH), lambda i: (0, i, 0)),
      ],
      out_shape=[
          jax.ShapeDtypeStruct((NCORE, N, HH), jnp.float32),
          jax.ShapeDtypeStruct((NCORE, N, HH), jnp.float32),
      ],
  )(nf, Wl, bl, W2, b2)


def _select_updated(agg_ref, xp_ref):
  a0 = agg_ref[0]
  a1 = agg_ref[1]
  has = a1[:, DEGC:DEGC + 1] > 0.0
  return jnp.concatenate(
      [jnp.where(has, a0, xp_ref[0]), jnp.where(has, a1, xp_ref[1])], axis=1)


def _tc_layer(agg, xp, W1, b1, W2n, b2n):
  def body(agg_ref, xp_ref, W1_ref, b1_ref, W2_ref, b2_ref, x_ref, h_ref):
    xm = _select_updated(agg_ref, xp_ref)
    xn = jnp.maximum(_dot(xm, W1_ref[...]) + b1_ref[...], 0.0)
    h = jnp.maximum(_dot(xn, W2_ref[...]) + b2_ref[...], 0.0)
    _split_store(x_ref, xn)
    _split_store(h_ref, h)

  return pl.pallas_call(
      body,
      grid=(N // NB,),
      in_specs=[
          pl.BlockSpec((NCORE, NB, HH), lambda i: (0, i, 0)),
          pl.BlockSpec((NCORE, NB, HH), lambda i: (0, i, 0)),
          pl.BlockSpec((HP, HP), lambda i: (0, 0)),
          pl.BlockSpec((1, HP), lambda i: (0, 0)),
          pl.BlockSpec((HP, HP), lambda i: (0, 0)),
          pl.BlockSpec((1, HP), lambda i: (0, 0)),
      ],
      out_specs=[
          pl.BlockSpec((NCORE, NB, HH), lambda i: (0, i, 0)),
          pl.BlockSpec((NCORE, NB, HH), lambda i: (0, i, 0)),
      ],
      out_shape=[
          jax.ShapeDtypeStruct((NCORE, N, HH), jnp.float32),
          jax.ShapeDtypeStruct((NCORE, N, HH), jnp.float32),
      ],
  )(agg, xp, W1, b1, W2n, b2n)


def _tc_final(agg, xp, W1, b1, Wr, br, onehot):
  def body(agg_ref, xp_ref, W1_ref, b1_ref, Wr_ref, br_ref, oh_ref, out_ref):
    i = pl.program_id(0)
    xm = _select_updated(agg_ref, xp_ref)
    x3 = jnp.maximum(_dot(xm, W1_ref[...]) + b1_ref[...], 0.0)
    nl = _dot(x3, Wr_ref[...]) + br_ref[...]
    contrib = _dot(oh_ref[...], nl)

    @pl.when(i == 0)
    def _():
      out_ref[...] = jnp.zeros_like(out_ref)

    out_ref[...] += contrib

  return pl.pallas_call(
      body,
      grid=(N // NB,),
      in_specs=[
          pl.BlockSpec((NCORE, NB, HH), lambda i: (0, i, 0)),
          pl.BlockSpec((NCORE, NB, HH), lambda i: (0, i, 0)),
          pl.BlockSpec((HP, HP), lambda i: (0, 0)),
          pl.BlockSpec((1, HP), lambda i: (0, 0)),
          pl.BlockSpec((HP, CPAD), lambda i: (0, 0)),
          pl.BlockSpec((1, CPAD), lambda i: (0, 0)),
          pl.BlockSpec((GP, NB), lambda i: (0, i)),
      ],
      out_specs=pl.BlockSpec((GP, CPAD), lambda i: (0, 0)),
      out_shape=jax.ShapeDtypeStruct((GP, CPAD), jnp.float32),
  )(agg, xp, W1, b1, Wr, br, onehot)


def kernel(node_feats, edge_index, graph_ids, W_lift, b_lift,
           W2_1, b2_1, W1_1, b1_1,
           W2_2, b2_2, W1_2, b1_2,
           W2_3, b2_3, W1_3, b1_3,
           W_read, b_read):
  f32 = jnp.float32
  h_dim = W2_1.shape[0]
  nf_p = jnp.pad(node_feats, ((0, 0), (0, DP - node_feats.shape[1])))
  Wl = jnp.pad(W_lift, ((0, DP - W_lift.shape[0]), (0, HP - W_lift.shape[1])))
  bl = jnp.pad(b_lift, (0, HP - b_lift.shape[0]))[None, :]

  def pad_w(w):
    return jnp.pad(w, ((0, HP - w.shape[0]), (0, HP - w.shape[1])))

  def pad_b(b, deg_one=False):
    bp = jnp.pad(b, (0, HP - b.shape[0]))
    if deg_one:
      bp = bp.at[h_dim].set(1.0)
    return bp[None, :]

  W2s = (pad_w(W2_1), pad_w(W2_2), pad_w(W2_3))
  b2s = (pad_b(b2_1, True), pad_b(b2_2, True), pad_b(b2_3, True))
  W1s = (pad_w(W1_1), pad_w(W1_2), pad_w(W1_3))
  b1s = (pad_b(b1_1), pad_b(b1_2), pad_b(b1_3))
  Wr = jnp.pad(W_read,
               ((0, HP - W_read.shape[0]), (0, CPAD - W_read.shape[1])))
  br = jnp.pad(b_read, (0, CPAD - b_read.shape[0]))[None, :]

  src = edge_index[0]
  dst = edge_index[1]
  src_off = jnp.reshape(jnp.stack([src, src + N]), (NCORE, NSUB, NCH, CH))
  dst3 = jnp.reshape(dst, (NSUB, NCH, CH))
  zeros_blk = jnp.zeros((RC, HH), f32)
  onehot = (graph_ids[None, :]
            == jnp.arange(GP, dtype=jnp.int32)[:, None]).astype(f32)

  x_st, h_st = _tc_lift(nf_p, Wl, bl, W2s[0], b2s[0])
  out = None
  for i in range(3):
    agg = _sc_edge_pass(jnp.reshape(h_st, (NCORE * N, HH)),
                        src_off, dst3, zeros_blk)
    if i < 2:
      x_st, h_st = _tc_layer(agg, x_st, W1s[i], b1s[i], W2s[i + 1], b2s[i + 1])
    else:
      out = _tc_final(agg, x_st, W1s[2], b1s[2], Wr, br, onehot)
  return out[:10, :W_read.shape[1]]


# speedup vs baseline: 2.0410x; 2.0410x over previous
import functools

import jax
import jax.numpy as jnp
from jax import lax
from jax.experimental import pallas as pl
from jax.experimental.pallas import tpu as pltpu
from jax.experimental.pallas import tpu_sc as plsc

N = 10000
E = 160000
DP = 128
SLAB = 128
NSLAB = 3
HP = SLAB * NSLAB
DEGC = 300 - 2 * SLAB
NCORE = 2
NSUB = 16
EPT = E // NSUB
EPT2 = E // (NSUB * NCORE)
CH = 40
NCHA = EPT // CH
NCHB = EPT2 // CH
RB = 624
ZC = 40
NB = 1000
GP = 16
CPAD = 128

_dot = functools.partial(jnp.dot, precision=lax.Precision.HIGHEST,
                         preferred_element_type=jnp.float32)


def _sc_edge_pass(h_slabs, src_all, dst_all, zeros_blk):
  mesh = plsc.VectorSubcoreMesh(core_axis_name="c", subcore_axis_name="s")

  @functools.partial(
      pl.kernel,
      out_type=jax.ShapeDtypeStruct((4, N, SLAB), jnp.float32),
      mesh=mesh,
      scratch_types=[
          pltpu.VMEM((EPT,), jnp.int32),
          pltpu.VMEM((CH,), jnp.int32),
          pltpu.VMEM((CH, SLAB), jnp.float32),
          pltpu.VMEM_SHARED((N, SLAB), jnp.float32),
          pltpu.SemaphoreType.DMA,
      ],
  )
  def k(h_hbm, src_hbm, dst_hbm, z_hbm, out_hbm,
        src_v, dstc_v, rows_v, agg_sh, sem):
    cid = lax.axis_index("c")
    sid = lax.axis_index("s")
    base = pl.multiple_of(sid * RB, 8)

    def zero_acc():
      for j in range(RB // ZC):
        pltpu.sync_copy(rows_v, agg_sh.at[pl.ds(base + j * ZC, ZC)])
      rem = RB - (RB // ZC) * ZC
      if rem:
        pltpu.sync_copy(rows_v.at[pl.ds(0, rem)],
                        agg_sh.at[pl.ds(base + RB - rem, rem)])

      @pl.when(sid == 0)
      def _():
        pltpu.sync_copy(rows_v.at[pl.ds(0, N - RB * NSUB)],
                        agg_sh.at[pl.ds(RB * NSUB, N - RB * NSUB)])

    def edge_loop(nch, src_base, dst_base):
      def body(kk, carry):
        off = pl.multiple_of(kk * CH, 8)
        pltpu.sync_copy(dst_hbm.at[pl.ds(dst_base + off, CH)], dstc_v)
        pltpu.async_copy(
            h_hbm.at[src_v.at[pl.ds(src_base + off, CH)]], rows_v, sem
        ).wait()
        pltpu.sync_copy(rows_v, agg_sh.at[dstc_v], add=True)
        return carry

      lax.fori_loop(0, nch, body, 0)

    def writeback(slot):
      for j in range(RB // ZC):
        pltpu.sync_copy(agg_sh.at[pl.ds(base + j * ZC, ZC)], rows_v)
        pltpu.sync_copy(rows_v, out_hbm.at[slot, pl.ds(base + j * ZC, ZC)])
      rem = RB - (RB // ZC) * ZC
      if rem:
        pltpu.sync_copy(agg_sh.at[pl.ds(base + RB - rem, rem)],
                        rows_v.at[pl.ds(0, rem)])
        pltpu.sync_copy(rows_v.at[pl.ds(0, rem)],
                        out_hbm.at[slot, pl.ds(base + RB - rem, rem)])

      @pl.when(sid == 0)
      def _():
        tail = N - RB * NSUB
        pltpu.sync_copy(agg_sh.at[pl.ds(RB * NSUB, tail)],
                        rows_v.at[pl.ds(0, tail)])
        pltpu.sync_copy(rows_v.at[pl.ds(0, tail)],
                        out_hbm.at[slot, pl.ds(RB * NSUB, tail)])

    pltpu.sync_copy(z_hbm, rows_v.at[pl.ds(0, ZC)])
    zero_acc()
    pltpu.sync_copy(
        src_hbm.at[pl.ds(cid * E + sid * EPT, EPT)], src_v)
    plsc.subcore_barrier()
    edge_loop(NCHA, 0, sid * EPT)
    plsc.subcore_barrier()
    writeback(cid)
    plsc.subcore_barrier()

    pltpu.sync_copy(z_hbm, rows_v.at[pl.ds(0, ZC)])
    zero_acc()
    pltpu.sync_copy(
        src_hbm.at[pl.ds(2 * E + (cid * NSUB + sid) * EPT2, EPT2)],
        src_v.at[pl.ds(0, EPT2)])
    plsc.subcore_barrier()
    edge_loop(NCHB, 0, (cid * NSUB + sid) * EPT2)
    plsc.subcore_barrier()
    writeback(2 + cid)

  return k(h_slabs, src_all, dst_all, zeros_blk)


def _split_store(ref, val):
  for s in range(NSLAB):
    ref[s] = val[:, s * SLAB:(s + 1) * SLAB]


def _tc_lift(nf, Wl, bl, W2, b2):
  def body(nf_ref, Wl_ref, bl_ref, W2_ref, b2_ref, x_ref, h_ref):
    x0 = _dot(nf_ref[...], Wl_ref[...]) + bl_ref[...]
    h1 = jnp.maximum(_dot(x0, W2_ref[...]) + b2_ref[...], 0.0)
    _split_store(x_ref, x0)
    _split_store(h_ref, h1)

  return pl.pallas_call(
      body,
      grid=(N // NB,),
      in_specs=[
          pl.BlockSpec((NB, DP), lambda i: (i, 0)),
          pl.BlockSpec((DP, HP), lambda i: (0, 0)),
          pl.BlockSpec((1, HP), lambda i: (0, 0)),
          pl.BlockSpec((HP, HP), lambda i: (0, 0)),
          pl.BlockSpec((1, HP), lambda i: (0, 0)),
      ],
      out_specs=[
          pl.BlockSpec((NSLAB, NB, SLAB), lambda i: (0, i, 0)),
          pl.BlockSpec((NSLAB, NB, SLAB), lambda i: (0, i, 0)),
      ],
      out_shape=[
          jax.ShapeDtypeStruct((NSLAB, N, SLAB), jnp.float32),
          jax.ShapeDtypeStruct((NSLAB, N, SLAB), jnp.float32),
      ],
  )(nf, Wl, bl, W2, b2)


def _select_updated(agg_ref, xp_ref):
  s2 = agg_ref[2] + agg_ref[3]
  has = s2[:, DEGC:DEGC + 1] > 0.0
  return jnp.concatenate(
      [jnp.where(has, agg_ref[0], xp_ref[0]),
       jnp.where(has, agg_ref[1], xp_ref[1]),
       jnp.where(has, s2, xp_ref[2])], axis=1)


def _tc_layer(agg, xp, W1, b1, W2n, b2n):
  def body(agg_ref, xp_ref, W1_ref, b1_ref, W2_ref, b2_ref, x_ref, h_ref):
    xm = _select_updated(agg_ref, xp_ref)
    xn = jnp.maximum(_dot(xm, W1_ref[...]) + b1_ref[...], 0.0)
    h = jnp.maximum(_dot(xn, W2_ref[...]) + b2_ref[...], 0.0)
    _split_store(x_ref, xn)
    _split_store(h_ref, h)

  return pl.pallas_call(
      body,
      grid=(N // NB,),
      in_specs=[
          pl.BlockSpec((4, NB, SLAB), lambda i: (0, i, 0)),
          pl.BlockSpec((NSLAB, NB, SLAB), lambda i: (0, i, 0)),
          pl.BlockSpec((HP, HP), lambda i: (0, 0)),
          pl.BlockSpec((1, HP), lambda i: (0, 0)),
          pl.BlockSpec((HP, HP), lambda i: (0, 0)),
          pl.BlockSpec((1, HP), lambda i: (0, 0)),
      ],
      out_specs=[
          pl.BlockSpec((NSLAB, NB, SLAB), lambda i: (0, i, 0)),
          pl.BlockSpec((NSLAB, NB, SLAB), lambda i: (0, i, 0)),
      ],
      out_shape=[
          jax.ShapeDtypeStruct((NSLAB, N, SLAB), jnp.float32),
          jax.ShapeDtypeStruct((NSLAB, N, SLAB), jnp.float32),
      ],
  )(agg, xp, W1, b1, W2n, b2n)


def _tc_final(agg, xp, W1, b1, Wr, br, onehot):
  def body(agg_ref, xp_ref, W1_ref, b1_ref, Wr_ref, br_ref, oh_ref, out_ref):
    i = pl.program_id(0)
    xm = _select_updated(agg_ref, xp_ref)
    x3 = jnp.maximum(_dot(xm, W1_ref[...]) + b1_ref[...], 0.0)
    nl = _dot(x3, Wr_ref[...]) + br_ref[...]
    contrib = _dot(oh_ref[0], nl)

    @pl.when(i == 0)
    def _():
      out_ref[...] = jnp.zeros_like(out_ref)

    out_ref[...] += contrib

  return pl.pallas_call(
      body,
      grid=(N // NB,),
      in_specs=[
          pl.BlockSpec((4, NB, SLAB), lambda i: (0, i, 0)),
          pl.BlockSpec((NSLAB, NB, SLAB), lambda i: (0, i, 0)),
          pl.BlockSpec((HP, HP), lambda i: (0, 0)),
          pl.BlockSpec((1, HP), lambda i: (0, 0)),
          pl.BlockSpec((HP, CPAD), lambda i: (0, 0)),
          pl.BlockSpec((1, CPAD), lambda i: (0, 0)),
          pl.BlockSpec((1, GP, NB), lambda i: (i, 0, 0)),
      ],
      out_specs=pl.BlockSpec((GP, CPAD), lambda i: (0, 0)),
      out_shape=jax.ShapeDtypeStruct((GP, CPAD), jnp.float32),
  )(agg, xp, W1, b1, Wr, br, onehot)


def kernel(node_feats, edge_index, graph_ids, W_lift, b_lift,
           W2_1, b2_1, W1_1, b1_1,
           W2_2, b2_2, W1_2, b1_2,
           W2_3, b2_3, W1_3, b1_3,
           W_read, b_read):
  f32 = jnp.float32
  h_dim = W2_1.shape[0]
  nf_p = jnp.pad(node_feats, ((0, 0), (0, DP - node_feats.shape[1])))
  Wl = jnp.pad(W_lift, ((0, DP - W_lift.shape[0]), (0, HP - W_lift.shape[1])))
  bl = jnp.pad(b_lift, (0, HP - b_lift.shape[0]))[None, :]

  def pad_w(w):
    return jnp.pad(w, ((0, HP - w.shape[0]), (0, HP - w.shape[1])))

  def pad_b(b, deg_one=False):
    bp = jnp.pad(b, (0, HP - b.shape[0]))
    if deg_one:
      bp = bp.at[h_dim].set(1.0)
    return bp[None, :]

  W2s = (pad_w(W2_1), pad_w(W2_2), pad_w(W2_3))
  b2s = (pad_b(b2_1, True), pad_b(b2_2, True), pad_b(b2_3, True))
  W1s = (pad_w(W1_1), pad_w(W1_2), pad_w(W1_3))
  b1s = (pad_b(b1_1), pad_b(b1_2), pad_b(b1_3))
  Wr = jnp.pad(W_read,
               ((0, HP - W_read.shape[0]), (0, CPAD - W_read.shape[1])))
  br = jnp.pad(b_read, (0, CPAD - b_read.shape[0]))[None, :]

  src = edge_index[0]
  dst = edge_index[1]
  src_all = jnp.reshape(
      jnp.stack([src, src + N, src + 2 * N]), (NSLAB * E,))
  zeros_blk = jnp.zeros((ZC, SLAB), f32)
  onehot = (graph_ids[None, :]
            == jnp.arange(GP, dtype=jnp.int32)[:, None]).astype(f32)
  onehot = jnp.transpose(jnp.reshape(onehot, (GP, N // NB, NB)), (1, 0, 2))

  x_st, h_st = _tc_lift(nf_p, Wl, bl, W2s[0], b2s[0])
  out = None
  for i in range(3):
    agg = _sc_edge_pass(jnp.reshape(h_st, (NSLAB * N, SLAB)),
                        src_all, dst, zeros_blk)
    if i < 2:
      x_st, h_st = _tc_layer(agg, x_st, W1s[i], b1s[i], W2s[i + 1], b2s[i + 1])
    else:
      out = _tc_final(agg, x_st, W1s[2], b1s[2], Wr, br, onehot)
  return out[:10, :W_read.shape[1]]


# speedup vs baseline: 5.7481x; 2.8163x over previous
import functools

import jax
import jax.numpy as jnp
from jax import lax
from jax.experimental import pallas as pl
from jax.experimental.pallas import tpu as pltpu
from jax.experimental.pallas import tpu_sc as plsc

N = 10000
E = 160000
DP = 128
SLAB = 128
NSLAB = 3
HP = SLAB * NSLAB
DEGC = 300 - 2 * SLAB
NCORE = 2
NSUB = 16
EPT = E // NSUB
EPT2 = E // (NSUB * NCORE)
CHA = 80
CHB = 40
NCHA = EPT // CHA
NCHB = EPT2 // CHB
RB = 624
ZC = 80
NB = 1000
GP = 16
CPAD = 128

_dot = functools.partial(jnp.dot, precision=lax.Precision.DEFAULT,
                         preferred_element_type=jnp.float32)
_dot_hi = functools.partial(jnp.dot, precision=lax.Precision.HIGHEST,
                            preferred_element_type=jnp.float32)


def _sc_edge_pass(h_slabs, src_all, dst_all, zeros_blk):
  mesh = plsc.VectorSubcoreMesh(core_axis_name="c", subcore_axis_name="s")

  @functools.partial(
      pl.kernel,
      out_type=jax.ShapeDtypeStruct((4, N, SLAB), jnp.float32),
      mesh=mesh,
      scratch_types=[
          pltpu.VMEM((EPT,), jnp.int32),
          pltpu.VMEM((CHA,), jnp.int32),
          pltpu.VMEM((CHA,), jnp.int32),
          pltpu.VMEM((CHA, SLAB), jnp.float32),
          pltpu.VMEM((CHA, SLAB), jnp.float32),
          pltpu.VMEM_SHARED((N, SLAB), jnp.float32),
          pltpu.SemaphoreType.DMA,
          pltpu.SemaphoreType.DMA,
          pltpu.SemaphoreType.DMA,
          pltpu.SemaphoreType.DMA,
      ],
  )
  def k(h_hbm, src_hbm, dst_hbm, z_hbm, out_hbm,
        src_v, dstc0_v, dstc1_v, rows0_v, rows1_v, agg_sh,
        semg0, semg1, semd0, semd1):
    rows_v = rows0_v
    cid = lax.axis_index("c")
    sid = lax.axis_index("s")
    base = pl.multiple_of(sid * RB, 8)

    def zero_acc():
      for j in range(RB // ZC):
        pltpu.sync_copy(rows_v, agg_sh.at[pl.ds(base + j * ZC, ZC)])
      rem = RB - (RB // ZC) * ZC
      if rem:
        pltpu.sync_copy(rows_v.at[pl.ds(0, rem)],
                        agg_sh.at[pl.ds(base + RB - rem, rem)])

      @pl.when(sid == 0)
      def _():
        pltpu.sync_copy(rows_v.at[pl.ds(0, N - RB * NSUB)],
                        agg_sh.at[pl.ds(RB * NSUB, N - RB * NSUB)])

    def edge_loop(ch, nch, src_base, dst_base):
      if ch == CHA:
        bufs = ((rows0_v, dstc0_v, semg0, semd0),
                (rows1_v, dstc1_v, semg1, semd1))
      else:
        bufs = ((rows0_v.at[pl.ds(0, ch)], dstc0_v.at[pl.ds(0, ch)],
                 semg0, semd0),
                (rows1_v.at[pl.ds(0, ch)], dstc1_v.at[pl.ds(0, ch)],
                 semg1, semd1))

      def issue(kk, buf):
        rows, dstc, sg, sd = buf
        off = pl.multiple_of(kk * ch, 8)
        pltpu.async_copy(dst_hbm.at[pl.ds(dst_base + off, ch)], dstc, sd)
        pltpu.async_copy(
            h_hbm.at[src_v.at[pl.ds(src_base + off, ch)]], rows, sg)

      def finish(kk, buf):
        rows, dstc, sg, sd = buf
        off = pl.multiple_of(kk * ch, 8)
        pltpu.make_async_copy(
            dst_hbm.at[pl.ds(dst_base + off, ch)], dstc, sd).wait()
        pltpu.make_async_copy(
            h_hbm.at[src_v.at[pl.ds(src_base + off, ch)]], rows, sg).wait()
        pltpu.sync_copy(rows, agg_sh.at[dstc], add=True)

      issue(0, bufs[0])

      def body(p, carry):
        k0 = 2 * p
        issue(k0 + 1, bufs[1])
        finish(k0, bufs[0])

        @pl.when(k0 + 2 < nch)
        def _():
          issue(k0 + 2, bufs[0])

        finish(k0 + 1, bufs[1])
        return carry

      lax.fori_loop(0, nch // 2, body, 0)
      if nch % 2:
        finish(nch - 1, bufs[0])

    def writeback(slot):
      for j in range(RB // ZC):
        pltpu.sync_copy(agg_sh.at[pl.ds(base + j * ZC, ZC)], rows_v)
        pltpu.sync_copy(rows_v, out_hbm.at[slot, pl.ds(base + j * ZC, ZC)])
      rem = RB - (RB // ZC) * ZC
      if rem:
        pltpu.sync_copy(agg_sh.at[pl.ds(base + RB - rem, rem)],
                        rows_v.at[pl.ds(0, rem)])
        pltpu.sync_copy(rows_v.at[pl.ds(0, rem)],
                        out_hbm.at[slot, pl.ds(base + RB - rem, rem)])

      @pl.when(sid == 0)
      def _():
        tail = N - RB * NSUB
        pltpu.sync_copy(agg_sh.at[pl.ds(RB * NSUB, tail)],
                        rows_v.at[pl.ds(0, tail)])
        pltpu.sync_copy(rows_v.at[pl.ds(0, tail)],
                        out_hbm.at[slot, pl.ds(RB * NSUB, tail)])

    pltpu.sync_copy(z_hbm, rows_v)
    zero_acc()
    pltpu.sync_copy(
        src_hbm.at[pl.ds(cid * E + sid * EPT, EPT)], src_v)
    plsc.subcore_barrier()
    edge_loop(CHA, NCHA, 0, sid * EPT)
    plsc.subcore_barrier()
    writeback(cid)
    plsc.subcore_barrier()

    pltpu.sync_copy(z_hbm, rows_v)
    zero_acc()
    pltpu.sync_copy(
        src_hbm.at[pl.ds(2 * E + (cid * NSUB + sid) * EPT2, EPT2)],
        src_v.at[pl.ds(0, EPT2)])
    plsc.subcore_barrier()
    edge_loop(CHB, NCHB, 0, (cid * NSUB + sid) * EPT2)
    plsc.subcore_barrier()
    writeback(2 + cid)

  return k(h_slabs, src_all, dst_all, zeros_blk)


def _split_store(ref, val):
  for s in range(NSLAB):
    ref[s] = val[:, s * SLAB:(s + 1) * SLAB]


def _tc_lift(nf, Wl, bl, W2, b2):
  def body(nf_ref, Wl_ref, bl_ref, W2_ref, b2_ref, x_ref, h_ref):
    x0 = _dot(nf_ref[...], Wl_ref[...]) + bl_ref[...]
    h1 = jnp.maximum(_dot(x0, W2_ref[...]) + b2_ref[...], 0.0)
    _split_store(x_ref, x0)
    _split_store(h_ref, h1)

  return pl.pallas_call(
      body,
      grid=(N // NB,),
      in_specs=[
          pl.BlockSpec((NB, DP), lambda i: (i, 0)),
          pl.BlockSpec((DP, HP), lambda i: (0, 0)),
          pl.BlockSpec((1, HP), lambda i: (0, 0)),
          pl.BlockSpec((HP, HP), lambda i: (0, 0)),
          pl.BlockSpec((1, HP), lambda i: (0, 0)),
      ],
      out_specs=[
          pl.BlockSpec((NSLAB, NB, SLAB), lambda i: (0, i, 0)),
          pl.BlockSpec((NSLAB, NB, SLAB), lambda i: (0, i, 0)),
      ],
      out_shape=[
          jax.ShapeDtypeStruct((NSLAB, N, SLAB), jnp.float32),
          jax.ShapeDtypeStruct((NSLAB, N, SLAB), jnp.float32),
      ],
  )(nf, Wl, bl, W2, b2)


def _select_updated(agg_ref, xp_ref):
  s2 = agg_ref[2] + agg_ref[3]
  has = s2[:, DEGC:DEGC + 1] > 0.0
  return jnp.concatenate(
      [jnp.where(has, agg_ref[0], xp_ref[0]),
       jnp.where(has, agg_ref[1], xp_ref[1]),
       jnp.where(has, s2, xp_ref[2])], axis=1)


def _tc_layer(agg, xp, W1, b1, W2n, b2n):
  def body(agg_ref, xp_ref, W1_ref, b1_ref, W2_ref, b2_ref, x_ref, h_ref):
    xm = _select_updated(agg_ref, xp_ref)
    xn = jnp.maximum(_dot(xm, W1_ref[...]) + b1_ref[...], 0.0)
    h = jnp.maximum(_dot(xn, W2_ref[...]) + b2_ref[...], 0.0)
    _split_store(x_ref, xn)
    _split_store(h_ref, h)

  return pl.pallas_call(
      body,
      grid=(N // NB,),
      in_specs=[
          pl.BlockSpec((4, NB, SLAB), lambda i: (0, i, 0)),
          pl.BlockSpec((NSLAB, NB, SLAB), lambda i: (0, i, 0)),
          pl.BlockSpec((HP, HP), lambda i: (0, 0)),
          pl.BlockSpec((1, HP), lambda i: (0, 0)),
          pl.BlockSpec((HP, HP), lambda i: (0, 0)),
          pl.BlockSpec((1, HP), lambda i: (0, 0)),
      ],
      out_specs=[
          pl.BlockSpec((NSLAB, NB, SLAB), lambda i: (0, i, 0)),
          pl.BlockSpec((NSLAB, NB, SLAB), lambda i: (0, i, 0)),
      ],
      out_shape=[
          jax.ShapeDtypeStruct((NSLAB, N, SLAB), jnp.float32),
          jax.ShapeDtypeStruct((NSLAB, N, SLAB), jnp.float32),
      ],
  )(agg, xp, W1, b1, W2n, b2n)


def _tc_final(agg, xp, W1, b1, Wr, br, onehot):
  def body(agg_ref, xp_ref, W1_ref, b1_ref, Wr_ref, br_ref, oh_ref, out_ref):
    i = pl.program_id(0)
    xm = _select_updated(agg_ref, xp_ref)
    x3 = jnp.maximum(_dot(xm, W1_ref[...]) + b1_ref[...], 0.0)
    nl = _dot(x3, Wr_ref[...]) + br_ref[...]
    contrib = _dot_hi(oh_ref[0], nl)

    @pl.when(i == 0)
    def _():
      out_ref[...] = jnp.zeros_like(out_ref)

    out_ref[...] += contrib

  return pl.pallas_call(
      body,
      grid=(N // NB,),
      in_specs=[
          pl.BlockSpec((4, NB, SLAB), lambda i: (0, i, 0)),
          pl.BlockSpec((NSLAB, NB, SLAB), lambda i: (0, i, 0)),
          pl.BlockSpec((HP, HP), lambda i: (0, 0)),
          pl.BlockSpec((1, HP), lambda i: (0, 0)),
          pl.BlockSpec((HP, CPAD), lambda i: (0, 0)),
          pl.BlockSpec((1, CPAD), lambda i: (0, 0)),
          pl.BlockSpec((1, GP, NB), lambda i: (i, 0, 0)),
      ],
      out_specs=pl.BlockSpec((GP, CPAD), lambda i: (0, 0)),
      out_shape=jax.ShapeDtypeStruct((GP, CPAD), jnp.float32),
  )(agg, xp, W1, b1, Wr, br, onehot)


def kernel(node_feats, edge_index, graph_ids, W_lift, b_lift,
           W2_1, b2_1, W1_1, b1_1,
           W2_2, b2_2, W1_2, b1_2,
           W2_3, b2_3, W1_3, b1_3,
           W_read, b_read):
  f32 = jnp.float32
  h_dim = W2_1.shape[0]
  nf_p = jnp.pad(node_feats, ((0, 0), (0, DP - node_feats.shape[1])))
  Wl = jnp.pad(W_lift, ((0, DP - W_lift.shape[0]), (0, HP - W_lift.shape[1])))
  bl = jnp.pad(b_lift, (0, HP - b_lift.shape[0]))[None, :]

  def pad_w(w):
    return jnp.pad(w, ((0, HP - w.shape[0]), (0, HP - w.shape[1])))

  def pad_b(b, deg_one=False):
    bp = jnp.pad(b, (0, HP - b.shape[0]))
    if deg_one:
      bp = bp.at[h_dim].set(1.0)
    return bp[None, :]

  W2s = (pad_w(W2_1), pad_w(W2_2), pad_w(W2_3))
  b2s = (pad_b(b2_1, True), pad_b(b2_2, True), pad_b(b2_3, True))
  W1s = (pad_w(W1_1), pad_w(W1_2), pad_w(W1_3))
  b1s = (pad_b(b1_1), pad_b(b1_2), pad_b(b1_3))
  Wr = jnp.pad(W_read,
               ((0, HP - W_read.shape[0]), (0, CPAD - W_read.shape[1])))
  br = jnp.pad(b_read, (0, CPAD - b_read.shape[0]))[None, :]

  src = edge_index[0]
  dst = edge_index[1]
  src_all = jnp.reshape(
      jnp.stack([src, src + N, src + 2 * N]), (NSLAB * E,))
  zeros_blk = jnp.zeros((ZC, SLAB), f32)
  onehot = (graph_ids[None, :]
            == jnp.arange(GP, dtype=jnp.int32)[:, None]).astype(f32)
  onehot = jnp.transpose(jnp.reshape(onehot, (GP, N // NB, NB)), (1, 0, 2))

  x_st, h_st = _tc_lift(nf_p, Wl, bl, W2s[0], b2s[0])
  out = None
  for i in range(3):
    agg = _sc_edge_pass(jnp.reshape(h_st, (NSLAB * N, SLAB)),
                        src_all, dst, zeros_blk)
    if i < 2:
      x_st, h_st = _tc_layer(agg, x_st, W1s[i], b1s[i], W2s[i + 1], b2s[i + 1])
    else:
      out = _tc_final(agg, x_st, W1s[2], b1s[2], Wr, br, onehot)
  return out[:10, :W_read.shape[1]]


# speedup vs baseline: 6.5362x; 1.1371x over previous
import functools

import jax
import jax.numpy as jnp
from jax import lax
from jax.experimental import pallas as pl
from jax.experimental.pallas import tpu as pltpu
from jax.experimental.pallas import tpu_sc as plsc

N = 10000
E = 160000
DP = 128
SLAB = 128
NSLAB = 3
HP = SLAB * NSLAB
DEGC = 300 - 2 * SLAB
NCORE = 2
NSUB = 16
EPT = E // NSUB
EPT2 = E // (NSUB * NCORE)
CHA = 80
CHB = 40
NCHA = EPT // CHA
NCHB = EPT2 // CHB
RB = 624
ZC = 80
NB = 1000
GP = 16
CPAD = 128

_dot = functools.partial(jnp.dot, precision=lax.Precision.DEFAULT,
                         preferred_element_type=jnp.float32)
_dot_hi = functools.partial(jnp.dot, precision=lax.Precision.HIGHEST,
                            preferred_element_type=jnp.float32)


def _sc_edge_pass(h_slabs, src_all, dst_all, zeros_blk):
  mesh = plsc.VectorSubcoreMesh(core_axis_name="c", subcore_axis_name="s")

  @functools.partial(
      pl.kernel,
      out_type=jax.ShapeDtypeStruct((4, N, SLAB), jnp.float32),
      mesh=mesh,
      scratch_types=(
          [pltpu.VMEM((CHA, SLAB), jnp.float32) for _ in range(3)]
          + [pltpu.VMEM((CHA,), jnp.int32) for _ in range(6)]
          + [pltpu.VMEM((CHA,), jnp.int32) for _ in range(6)]
          + [pltpu.VMEM_SHARED((N, SLAB), jnp.float32)]
          + [pltpu.SemaphoreType.DMA for _ in range(12)]
      ),
  )
  def k(h_hbm, src_hbm, dst_hbm, z_hbm, out_hbm, *scratch):
    rows_b = scratch[0:3]
    srcc_b = scratch[3:9]
    dstc_b = scratch[9:15]
    agg_sh = scratch[15]
    semg = scratch[16:19]
    semi = scratch[19:25]
    sems = scratch[25:28]
    rows_v = rows_b[0]
    cid = lax.axis_index("c")
    sid = lax.axis_index("s")
    base = pl.multiple_of(sid * RB, 8)

    def zero_acc():
      for j in range(RB // ZC):
        pltpu.sync_copy(rows_v, agg_sh.at[pl.ds(base + j * ZC, ZC)])
      rem = RB - (RB // ZC) * ZC
      if rem:
        pltpu.sync_copy(rows_v.at[pl.ds(0, rem)],
                        agg_sh.at[pl.ds(base + RB - rem, rem)])

      @pl.when(sid == 0)
      def _():
        pltpu.sync_copy(rows_v.at[pl.ds(0, N - RB * NSUB)],
                        agg_sh.at[pl.ds(RB * NSUB, N - RB * NSUB)])

    def edge_loop(ch, nch, src_base, dst_base):
      if ch == CHA:
        rows = list(rows_b)
        srcc = list(srcc_b)
        dstc = list(dstc_b)
      else:
        rows = [r.at[pl.ds(0, ch)] for r in rows_b]
        srcc = [r.at[pl.ds(0, ch)] for r in srcc_b]
        dstc = [r.at[pl.ds(0, ch)] for r in dstc_b]

      def idx_issue(kk, r):
        off = pl.multiple_of(kk * ch, 8)
        pltpu.async_copy(src_hbm.at[pl.ds(src_base + off, ch)],
                         srcc[r], semi[r])
        pltpu.async_copy(dst_hbm.at[pl.ds(dst_base + off, ch)],
                         dstc[r], semi[r])

      def idx_wait(kk, r):
        off = pl.multiple_of(kk * ch, 8)
        pltpu.make_async_copy(src_hbm.at[pl.ds(src_base + off, ch)],
                              srcc[r], semi[r]).wait()
        pltpu.make_async_copy(dst_hbm.at[pl.ds(dst_base + off, ch)],
                              dstc[r], semi[r]).wait()

      def gather_issue(r3, r6):
        pltpu.async_copy(h_hbm.at[srcc[r6]], rows[r3], semg[r3])

      def gather_wait(r3, r6):
        pltpu.make_async_copy(h_hbm.at[srcc[r6]], rows[r3], semg[r3]).wait()

      def scatter_issue(r3, r6):
        pltpu.async_copy(rows[r3], agg_sh.at[dstc[r6]], sems[r3], add=True)

      def scatter_wait(r3, r6):
        pltpu.make_async_copy(rows[r3], agg_sh.at[dstc[r6]],
                              sems[r3]).wait()

      for r in range(3):
        idx_issue(r, r)
      idx_wait(0, 0)
      gather_issue(0, 0)
      idx_wait(1, 1)
      gather_issue(1, 1)

      def body(p, carry):
        k0 = 6 * p
        for r in range(6):
          kk = k0 + r

          @pl.when(kk < nch)
          def _(kk=kk, r=r):
            gather_wait(r % 3, r)
            scatter_issue(r % 3, r)

            @pl.when(kk >= 1)
            def _():
              scatter_wait((r - 1) % 3, (r - 1) % 6)

            @pl.when(kk + 3 < nch)
            def _():
              idx_issue(kk + 3, (r + 3) % 6)

            @pl.when(kk + 2 < nch)
            def _():
              idx_wait(kk + 2, (r + 2) % 6)
              gather_issue((r + 2) % 3, (r + 2) % 6)

        return carry

      lax.fori_loop(0, (nch + 5) // 6, body, 0)
      scatter_wait((nch - 1) % 3, (nch - 1) % 6)

    def writeback(slot):
      for j in range(RB // ZC):
        pltpu.sync_copy(agg_sh.at[pl.ds(base + j * ZC, ZC)], rows_v)
        pltpu.sync_copy(rows_v, out_hbm.at[slot, pl.ds(base + j * ZC, ZC)])
      rem = RB - (RB // ZC) * ZC
      if rem:
        pltpu.sync_copy(agg_sh.at[pl.ds(base + RB - rem, rem)],
                        rows_v.at[pl.ds(0, rem)])
        pltpu.sync_copy(rows_v.at[pl.ds(0, rem)],
                        out_hbm.at[slot, pl.ds(base + RB - rem, rem)])

      @pl.when(sid == 0)
      def _():
        tail = N - RB * NSUB
        pltpu.sync_copy(agg_sh.at[pl.ds(RB * NSUB, tail)],
                        rows_v.at[pl.ds(0, tail)])
        pltpu.sync_copy(rows_v.at[pl.ds(0, tail)],
                        out_hbm.at[slot, pl.ds(RB * NSUB, tail)])

    pltpu.sync_copy(z_hbm, rows_v)
    zero_acc()
    plsc.subcore_barrier()
    edge_loop(CHA, NCHA, cid * E + sid * EPT, sid * EPT)
    plsc.subcore_barrier()
    writeback(cid)
    plsc.subcore_barrier()

    pltpu.sync_copy(z_hbm, rows_v)
    zero_acc()
    plsc.subcore_barrier()
    edge_loop(CHB, NCHB, 2 * E + (cid * NSUB + sid) * EPT2,
              (cid * NSUB + sid) * EPT2)
    plsc.subcore_barrier()
    writeback(2 + cid)

  return k(h_slabs, src_all, dst_all, zeros_blk)


def _split_store(ref, val):
  for s in range(NSLAB):
    ref[s] = val[:, s * SLAB:(s + 1) * SLAB]


def _tc_lift(nf, Wl, bl, W2, b2):
  def body(nf_ref, Wl_ref, bl_ref, W2_ref, b2_ref, x_ref, h_ref):
    x0 = _dot(nf_ref[...], Wl_ref[...]) + bl_ref[...]
    h1 = jnp.maximum(_dot(x0, W2_ref[...]) + b2_ref[...], 0.0)
    _split_store(x_ref, x0)
    _split_store(h_ref, h1)

  return pl.pallas_call(
      body,
      grid=(N // NB,),
      in_specs=[
          pl.BlockSpec((NB, DP), lambda i: (i, 0)),
          pl.BlockSpec((DP, HP), lambda i: (0, 0)),
          pl.BlockSpec((1, HP), lambda i: (0, 0)),
          pl.BlockSpec((HP, HP), lambda i: (0, 0)),
          pl.BlockSpec((1, HP), lambda i: (0, 0)),
      ],
      out_specs=[
          pl.BlockSpec((NSLAB, NB, SLAB), lambda i: (0, i, 0)),
          pl.BlockSpec((NSLAB, NB, SLAB), lambda i: (0, i, 0)),
      ],
      out_shape=[
          jax.ShapeDtypeStruct((NSLAB, N, SLAB), jnp.float32),
          jax.ShapeDtypeStruct((NSLAB, N, SLAB), jnp.float32),
      ],
  )(nf, Wl, bl, W2, b2)


def _select_updated(agg_ref, xp_ref):
  s2 = agg_ref[2] + agg_ref[3]
  has = s2[:, DEGC:DEGC + 1] > 0.0
  return jnp.concatenate(
      [jnp.where(has, agg_ref[0], xp_ref[0]),
       jnp.where(has, agg_ref[1], xp_ref[1]),
       jnp.where(has, s2, xp_ref[2])], axis=1)


def _tc_layer(agg, xp, W1, b1, W2n, b2n):
  def body(agg_ref, xp_ref, W1_ref, b1_ref, W2_ref, b2_ref, x_ref, h_ref):
    xm = _select_updated(agg_ref, xp_ref)
    xn = jnp.maximum(_dot(xm, W1_ref[...]) + b1_ref[...], 0.0)
    h = jnp.maximum(_dot(xn, W2_ref[...]) + b2_ref[...], 0.0)
    _split_store(x_ref, xn)
    _split_store(h_ref, h)

  return pl.pallas_call(
      body,
      grid=(N // NB,),
      in_specs=[
          pl.BlockSpec((4, NB, SLAB), lambda i: (0, i, 0)),
          pl.BlockSpec((NSLAB, NB, SLAB), lambda i: (0, i, 0)),
          pl.BlockSpec((HP, HP), lambda i: (0, 0)),
          pl.BlockSpec((1, HP), lambda i: (0, 0)),
          pl.BlockSpec((HP, HP), lambda i: (0, 0)),
          pl.BlockSpec((1, HP), lambda i: (0, 0)),
      ],
      out_specs=[
          pl.BlockSpec((NSLAB, NB, SLAB), lambda i: (0, i, 0)),
          pl.BlockSpec((NSLAB, NB, SLAB), lambda i: (0, i, 0)),
      ],
      out_shape=[
          jax.ShapeDtypeStruct((NSLAB, N, SLAB), jnp.float32),
          jax.ShapeDtypeStruct((NSLAB, N, SLAB), jnp.float32),
      ],
  )(agg, xp, W1, b1, W2n, b2n)


def _tc_final(agg, xp, W1, b1, Wr, br, onehot):
  def body(agg_ref, xp_ref, W1_ref, b1_ref, Wr_ref, br_ref, oh_ref, out_ref):
    i = pl.program_id(0)
    xm = _select_updated(agg_ref, xp_ref)
    x3 = jnp.maximum(_dot(xm, W1_ref[...]) + b1_ref[...], 0.0)
    nl = _dot(x3, Wr_ref[...]) + br_ref[...]
    contrib = _dot_hi(oh_ref[0], nl)

    @pl.when(i == 0)
    def _():
      out_ref[...] = jnp.zeros_like(out_ref)

    out_ref[...] += contrib

  return pl.pallas_call(
      body,
      grid=(N // NB,),
      in_specs=[
          pl.BlockSpec((4, NB, SLAB), lambda i: (0, i, 0)),
          pl.BlockSpec((NSLAB, NB, SLAB), lambda i: (0, i, 0)),
          pl.BlockSpec((HP, HP), lambda i: (0, 0)),
          pl.BlockSpec((1, HP), lambda i: (0, 0)),
          pl.BlockSpec((HP, CPAD), lambda i: (0, 0)),
          pl.BlockSpec((1, CPAD), lambda i: (0, 0)),
          pl.BlockSpec((1, GP, NB), lambda i: (i, 0, 0)),
      ],
      out_specs=pl.BlockSpec((GP, CPAD), lambda i: (0, 0)),
      out_shape=jax.ShapeDtypeStruct((GP, CPAD), jnp.float32),
  )(agg, xp, W1, b1, Wr, br, onehot)


def kernel(node_feats, edge_index, graph_ids, W_lift, b_lift,
           W2_1, b2_1, W1_1, b1_1,
           W2_2, b2_2, W1_2, b1_2,
           W2_3, b2_3, W1_3, b1_3,
           W_read, b_read):
  f32 = jnp.float32
  h_dim = W2_1.shape[0]
  nf_p = jnp.pad(node_feats, ((0, 0), (0, DP - node_feats.shape[1])))
  Wl = jnp.pad(W_lift, ((0, DP - W_lift.shape[0]), (0, HP - W_lift.shape[1])))
  bl = jnp.pad(b_lift, (0, HP - b_lift.shape[0]))[None, :]

  def pad_w(w):
    return jnp.pad(w, ((0, HP - w.shape[0]), (0, HP - w.shape[1])))

  def pad_b(b, deg_one=False):
    bp = jnp.pad(b, (0, HP - b.shape[0]))
    if deg_one:
      bp = bp.at[h_dim].set(1.0)
    return bp[None, :]

  W2s = (pad_w(W2_1), pad_w(W2_2), pad_w(W2_3))
  b2s = (pad_b(b2_1, True), pad_b(b2_2, True), pad_b(b2_3, True))
  W1s = (pad_w(W1_1), pad_w(W1_2), pad_w(W1_3))
  b1s = (pad_b(b1_1), pad_b(b1_2), pad_b(b1_3))
  Wr = jnp.pad(W_read,
               ((0, HP - W_read.shape[0]), (0, CPAD - W_read.shape[1])))
  br = jnp.pad(b_read, (0, CPAD - b_read.shape[0]))[None, :]

  src = edge_index[0]
  dst = edge_index[1]
  src_all = jnp.reshape(
      jnp.stack([src, src + N, src + 2 * N]), (NSLAB * E,))
  zeros_blk = jnp.zeros((ZC, SLAB), f32)
  onehot = (graph_ids[None, :]
            == jnp.arange(GP, dtype=jnp.int32)[:, None]).astype(f32)
  onehot = jnp.transpose(jnp.reshape(onehot, (GP, N // NB, NB)), (1, 0, 2))

  x_st, h_st = _tc_lift(nf_p, Wl, bl, W2s[0], b2s[0])
  out = None
  for i in range(3):
    agg = _sc_edge_pass(jnp.reshape(h_st, (NSLAB * N, SLAB)),
                        src_all, dst, zeros_blk)
    if i < 2:
      x_st, h_st = _tc_layer(agg, x_st, W1s[i], b1s[i], W2s[i + 1], b2s[i + 1])
    else:
      out = _tc_final(agg, x_st, W1s[2], b1s[2], Wr, br, onehot)
  return out[:10, :W_read.shape[1]]


# speedup vs baseline: 6.6083x; 1.0110x over previous
import functools

import jax
import jax.numpy as jnp
from jax import lax
from jax.experimental import pallas as pl
from jax.experimental.pallas import tpu as pltpu
from jax.experimental.pallas import tpu_sc as plsc

N = 10000
E = 160000
DP = 128
SLAB = 128
NSLAB = 3
HP = SLAB * NSLAB
DEGC = 300 - 2 * SLAB
NCORE = 2
NSUB = 16
EPT = E // NSUB
EPT2 = E // (NSUB * NCORE)
CHA = 80
CHB = 40
NCHA = EPT // CHA
NCHB = EPT2 // CHB
RB = 624
ZC = 80
NB = 1000
GP = 16
CPAD = 128

_dot = functools.partial(jnp.dot, precision=lax.Precision.DEFAULT,
                         preferred_element_type=jnp.float32)
_dot_hi = functools.partial(jnp.dot, precision=lax.Precision.HIGHEST,
                            preferred_element_type=jnp.float32)


def _sc_edge_pass(h_slabs, src_all, dst_all, zeros_blk):
  mesh = plsc.VectorSubcoreMesh(core_axis_name="c", subcore_axis_name="s")

  @functools.partial(
      pl.kernel,
      out_type=jax.ShapeDtypeStruct((4, N, SLAB), jnp.float32),
      mesh=mesh,
      scratch_types=(
          [pltpu.VMEM((CHA, SLAB), jnp.float32) for _ in range(3)]
          + [pltpu.VMEM((CHA,), jnp.int32) for _ in range(6)]
          + [pltpu.VMEM((CHA,), jnp.int32) for _ in range(6)]
          + [pltpu.VMEM_SHARED((N, SLAB), jnp.float32)]
          + [pltpu.SemaphoreType.DMA for _ in range(12)]
      ),
  )
  def k(h_hbm, src_hbm, dst_hbm, z_hbm, out_hbm, *scratch):
    rows_b = scratch[0:3]
    srcc_b = scratch[3:9]
    dstc_b = scratch[9:15]
    agg_sh = scratch[15]
    semg = scratch[16:19]
    semi = scratch[19:25]
    sems = scratch[25:28]
    rows_v = rows_b[0]
    cid = lax.axis_index("c")
    sid = lax.axis_index("s")
    base = pl.multiple_of(sid * RB, 8)

    def zero_acc():
      for j in range(RB // ZC):
        pltpu.async_copy(rows_v, agg_sh.at[pl.ds(base + j * ZC, ZC)], sems[0])
      rem = RB - (RB // ZC) * ZC
      if rem:
        pltpu.async_copy(rows_v.at[pl.ds(0, rem)],
                         agg_sh.at[pl.ds(base + RB - rem, rem)], sems[1])

      @pl.when(sid == 0)
      def _():
        pltpu.async_copy(rows_v.at[pl.ds(0, N - RB * NSUB)],
                         agg_sh.at[pl.ds(RB * NSUB, N - RB * NSUB)], sems[2])

      for j in range(RB // ZC):
        pltpu.make_async_copy(rows_v, agg_sh.at[pl.ds(base + j * ZC, ZC)],
                              sems[0]).wait()
      if rem:
        pltpu.make_async_copy(rows_v.at[pl.ds(0, rem)],
                              agg_sh.at[pl.ds(base + RB - rem, rem)],
                              sems[1]).wait()

      @pl.when(sid == 0)
      def _():
        pltpu.make_async_copy(rows_v.at[pl.ds(0, N - RB * NSUB)],
                              agg_sh.at[pl.ds(RB * NSUB, N - RB * NSUB)],
                              sems[2]).wait()

    def edge_loop(ch, nch, src_base, dst_base):
      if ch == CHA:
        rows = list(rows_b)
        srcc = list(srcc_b)
        dstc = list(dstc_b)
      else:
        rows = [r.at[pl.ds(0, ch)] for r in rows_b]
        srcc = [r.at[pl.ds(0, ch)] for r in srcc_b]
        dstc = [r.at[pl.ds(0, ch)] for r in dstc_b]

      def idx_issue(kk, r):
        off = pl.multiple_of(kk * ch, 8)
        pltpu.async_copy(src_hbm.at[pl.ds(src_base + off, ch)],
                         srcc[r], semi[r])
        pltpu.async_copy(dst_hbm.at[pl.ds(dst_base + off, ch)],
                         dstc[r], semi[r])

      def idx_wait(kk, r):
        off = pl.multiple_of(kk * ch, 8)
        pltpu.make_async_copy(src_hbm.at[pl.ds(src_base + off, ch)],
                              srcc[r], semi[r]).wait()
        pltpu.make_async_copy(dst_hbm.at[pl.ds(dst_base + off, ch)],
                              dstc[r], semi[r]).wait()

      def gather_issue(r3, r6):
        pltpu.async_copy(h_hbm.at[srcc[r6]], rows[r3], semg[r3])

      def gather_wait(r3, r6):
        pltpu.make_async_copy(h_hbm.at[srcc[r6]], rows[r3], semg[r3]).wait()

      def scatter_issue(r3, r6):
        pltpu.async_copy(rows[r3], agg_sh.at[dstc[r6]], sems[r3], add=True)

      def scatter_wait(r3, r6):
        pltpu.make_async_copy(rows[r3], agg_sh.at[dstc[r6]],
                              sems[r3]).wait()

      for r in range(3):
        idx_issue(r, r)
      idx_wait(0, 0)
      gather_issue(0, 0)
      idx_wait(1, 1)
      gather_issue(1, 1)

      def body(p, carry):
        k0 = 6 * p
        for r in range(6):
          kk = k0 + r

          @pl.when(kk < nch)
          def _(kk=kk, r=r):
            gather_wait(r % 3, r)
            scatter_issue(r % 3, r)

            @pl.when(kk >= 1)
            def _():
              scatter_wait((r - 1) % 3, (r - 1) % 6)

            @pl.when(kk + 3 < nch)
            def _():
              idx_issue(kk + 3, (r + 3) % 6)

            @pl.when(kk + 2 < nch)
            def _():
              idx_wait(kk + 2, (r + 2) % 6)
              gather_issue((r + 2) % 3, (r + 2) % 6)

        return carry

      lax.fori_loop(0, (nch + 5) // 6, body, 0)
      scatter_wait((nch - 1) % 3, (nch - 1) % 6)

    def writeback(slot):
      for j in range(RB // ZC):
        pltpu.async_copy(agg_sh.at[pl.ds(base + j * ZC, ZC)],
                         out_hbm.at[slot, pl.ds(base + j * ZC, ZC)], semg[0])
      rem = RB - (RB // ZC) * ZC
      if rem:
        pltpu.async_copy(agg_sh.at[pl.ds(base + RB - rem, rem)],
                         out_hbm.at[slot, pl.ds(base + RB - rem, rem)],
                         semg[1])

      @pl.when(sid == 0)
      def _():
        tail = N - RB * NSUB
        pltpu.async_copy(agg_sh.at[pl.ds(RB * NSUB, tail)],
                         out_hbm.at[slot, pl.ds(RB * NSUB, tail)], semg[2])

      for j in range(RB // ZC):
        pltpu.make_async_copy(agg_sh.at[pl.ds(base + j * ZC, ZC)],
                              out_hbm.at[slot, pl.ds(base + j * ZC, ZC)],
                              semg[0]).wait()
      if rem:
        pltpu.make_async_copy(agg_sh.at[pl.ds(base + RB - rem, rem)],
                              out_hbm.at[slot, pl.ds(base + RB - rem, rem)],
                              semg[1]).wait()

      @pl.when(sid == 0)
      def _():
        tail = N - RB * NSUB
        pltpu.make_async_copy(agg_sh.at[pl.ds(RB * NSUB, tail)],
                              out_hbm.at[slot, pl.ds(RB * NSUB, tail)],
                              semg[2]).wait()

    pltpu.sync_copy(z_hbm, rows_v)
    zero_acc()
    plsc.subcore_barrier()
    edge_loop(CHA, NCHA, cid * E + sid * EPT, sid * EPT)
    plsc.subcore_barrier()
    writeback(cid)
    plsc.subcore_barrier()

    pltpu.sync_copy(z_hbm, rows_v)
    zero_acc()
    plsc.subcore_barrier()
    edge_loop(CHB, NCHB, 2 * E + (cid * NSUB + sid) * EPT2,
              (cid * NSUB + sid) * EPT2)
    plsc.subcore_barrier()
    writeback(2 + cid)

  return k(h_slabs, src_all, dst_all, zeros_blk)


def _split_store(ref, val):
  for s in range(NSLAB):
    ref[s] = val[:, s * SLAB:(s + 1) * SLAB]


def _tc_lift(nf, Wl, bl, W2, b2):
  def body(nf_ref, Wl_ref, bl_ref, W2_ref, b2_ref, x_ref, h_ref):
    x0 = _dot(nf_ref[...], Wl_ref[...]) + bl_ref[...]
    h1 = jnp.maximum(_dot(x0, W2_ref[...]) + b2_ref[...], 0.0)
    _split_store(x_ref, x0)
    _split_store(h_ref, h1)

  return pl.pallas_call(
      body,
      grid=(N // NB,),
      in_specs=[
          pl.BlockSpec((NB, DP), lambda i: (i, 0)),
          pl.BlockSpec((DP, HP), lambda i: (0, 0)),
          pl.BlockSpec((1, HP), lambda i: (0, 0)),
          pl.BlockSpec((HP, HP), lambda i: (0, 0)),
          pl.BlockSpec((1, HP), lambda i: (0, 0)),
      ],
      out_specs=[
          pl.BlockSpec((NSLAB, NB, SLAB), lambda i: (0, i, 0)),
          pl.BlockSpec((NSLAB, NB, SLAB), lambda i: (0, i, 0)),
      ],
      out_shape=[
          jax.ShapeDtypeStruct((NSLAB, N, SLAB), jnp.float32),
          jax.ShapeDtypeStruct((NSLAB, N, SLAB), jnp.float32),
      ],
  )(nf, Wl, bl, W2, b2)


def _select_updated(agg_ref, xp_ref):
  s2 = agg_ref[2] + agg_ref[3]
  has = s2[:, DEGC:DEGC + 1] > 0.0
  return jnp.concatenate(
      [jnp.where(has, agg_ref[0], xp_ref[0]),
       jnp.where(has, agg_ref[1], xp_ref[1]),
       jnp.where(has, s2, xp_ref[2])], axis=1)


def _tc_layer(agg, xp, W1, b1, W2n, b2n):
  def body(agg_ref, xp_ref, W1_ref, b1_ref, W2_ref, b2_ref, x_ref, h_ref):
    xm = _select_updated(agg_ref, xp_ref)
    xn = jnp.maximum(_dot(xm, W1_ref[...]) + b1_ref[...], 0.0)
    h = jnp.maximum(_dot(xn, W2_ref[...]) + b2_ref[...], 0.0)
    _split_store(x_ref, xn)
    _split_store(h_ref, h)

  return pl.pallas_call(
      body,
      grid=(N // NB,),
      in_specs=[
          pl.BlockSpec((4, NB, SLAB), lambda i: (0, i, 0)),
          pl.BlockSpec((NSLAB, NB, SLAB), lambda i: (0, i, 0)),
          pl.BlockSpec((HP, HP), lambda i: (0, 0)),
          pl.BlockSpec((1, HP), lambda i: (0, 0)),
          pl.BlockSpec((HP, HP), lambda i: (0, 0)),
          pl.BlockSpec((1, HP), lambda i: (0, 0)),
      ],
      out_specs=[
          pl.BlockSpec((NSLAB, NB, SLAB), lambda i: (0, i, 0)),
          pl.BlockSpec((NSLAB, NB, SLAB), lambda i: (0, i, 0)),
      ],
      out_shape=[
          jax.ShapeDtypeStruct((NSLAB, N, SLAB), jnp.float32),
          jax.ShapeDtypeStruct((NSLAB, N, SLAB), jnp.float32),
      ],
  )(agg, xp, W1, b1, W2n, b2n)


def _tc_final(agg, xp, W1, b1, Wr, br, onehot):
  def body(agg_ref, xp_ref, W1_ref, b1_ref, Wr_ref, br_ref, oh_ref, out_ref):
    i = pl.program_id(0)
    xm = _select_updated(agg_ref, xp_ref)
    x3 = jnp.maximum(_dot(xm, W1_ref[...]) + b1_ref[...], 0.0)
    nl = _dot(x3, Wr_ref[...]) + br_ref[...]
    contrib = _dot_hi(oh_ref[0], nl)

    @pl.when(i == 0)
    def _():
      out_ref[...] = jnp.zeros_like(out_ref)

    out_ref[...] += contrib

  return pl.pallas_call(
      body,
      grid=(N // NB,),
      in_specs=[
          pl.BlockSpec((4, NB, SLAB), lambda i: (0, i, 0)),
          pl.BlockSpec((NSLAB, NB, SLAB), lambda i: (0, i, 0)),
          pl.BlockSpec((HP, HP), lambda i: (0, 0)),
          pl.BlockSpec((1, HP), lambda i: (0, 0)),
          pl.BlockSpec((HP, CPAD), lambda i: (0, 0)),
          pl.BlockSpec((1, CPAD), lambda i: (0, 0)),
          pl.BlockSpec((1, GP, NB), lambda i: (i, 0, 0)),
      ],
      out_specs=pl.BlockSpec((GP, CPAD), lambda i: (0, 0)),
      out_shape=jax.ShapeDtypeStruct((GP, CPAD), jnp.float32),
  )(agg, xp, W1, b1, Wr, br, onehot)


def kernel(node_feats, edge_index, graph_ids, W_lift, b_lift,
           W2_1, b2_1, W1_1, b1_1,
           W2_2, b2_2, W1_2, b1_2,
           W2_3, b2_3, W1_3, b1_3,
           W_read, b_read):
  f32 = jnp.float32
  h_dim = W2_1.shape[0]
  nf_p = jnp.pad(node_feats, ((0, 0), (0, DP - node_feats.shape[1])))
  Wl = jnp.pad(W_lift, ((0, DP - W_lift.shape[0]), (0, HP - W_lift.shape[1])))
  bl = jnp.pad(b_lift, (0, HP - b_lift.shape[0]))[None, :]

  def pad_w(w):
    return jnp.pad(w, ((0, HP - w.shape[0]), (0, HP - w.shape[1])))

  def pad_b(b, deg_one=False):
    bp = jnp.pad(b, (0, HP - b.shape[0]))
    if deg_one:
      bp = bp.at[h_dim].set(1.0)
    return bp[None, :]

  W2s = (pad_w(W2_1), pad_w(W2_2), pad_w(W2_3))
  b2s = (pad_b(b2_1, True), pad_b(b2_2, True), pad_b(b2_3, True))
  W1s = (pad_w(W1_1), pad_w(W1_2), pad_w(W1_3))
  b1s = (pad_b(b1_1), pad_b(b1_2), pad_b(b1_3))
  Wr = jnp.pad(W_read,
               ((0, HP - W_read.shape[0]), (0, CPAD - W_read.shape[1])))
  br = jnp.pad(b_read, (0, CPAD - b_read.shape[0]))[None, :]

  src = edge_index[0]
  dst = edge_index[1]
  src_all = jnp.reshape(
      jnp.stack([src, src + N, src + 2 * N]), (NSLAB * E,))
  zeros_blk = jnp.zeros((ZC, SLAB), f32)
  onehot = (graph_ids[None, :]
            == jnp.arange(GP, dtype=jnp.int32)[:, None]).astype(f32)
  onehot = jnp.transpose(jnp.reshape(onehot, (GP, N // NB, NB)), (1, 0, 2))

  x_st, h_st = _tc_lift(nf_p, Wl, bl, W2s[0], b2s[0])
  out = None
  for i in range(3):
    agg = _sc_edge_pass(jnp.reshape(h_st, (NSLAB * N, SLAB)),
                        src_all, dst, zeros_blk)
    if i < 2:
      x_st, h_st = _tc_layer(agg, x_st, W1s[i], b1s[i], W2s[i + 1], b2s[i + 1])
    else:
      out = _tc_final(agg, x_st, W1s[2], b1s[2], Wr, br, onehot)
  return out[:10, :W_read.shape[1]]


# speedup vs baseline: 6.6246x; 1.0025x over previous
import functools

import jax
import jax.numpy as jnp
from jax import lax
from jax.experimental import pallas as pl
from jax.experimental.pallas import tpu as pltpu
from jax.experimental.pallas import tpu_sc as plsc

N = 10000
E = 160000
DP = 128
SLAB = 128
NSLAB = 3
HP = SLAB * NSLAB
DEGC = 300 - 2 * SLAB
NCORE = 2
NSUB = 16
EPT = E // NSUB
EPT2 = E // (NSUB * NCORE)
CHA = 80
CHB = 40
NCHA = EPT // CHA
NCHB = EPT2 // CHB
RB = 624
ZC = 80
NB = 1000
GP = 16
CPAD = 128

_dot = functools.partial(jnp.dot, precision=lax.Precision.DEFAULT,
                         preferred_element_type=jnp.float32)
_dot_hi = functools.partial(jnp.dot, precision=lax.Precision.HIGHEST,
                            preferred_element_type=jnp.float32)


def _sc_edge_pass(h_slabs, src_all, dst_all, zeros_blk):
  mesh = plsc.VectorSubcoreMesh(core_axis_name="c", subcore_axis_name="s")

  @functools.partial(
      pl.kernel,
      out_type=jax.ShapeDtypeStruct((4, N, SLAB), jnp.float32),
      mesh=mesh,
      scratch_types=(
          [pltpu.VMEM((CHA, SLAB), jnp.float32) for _ in range(3)]
          + [pltpu.VMEM((CHA,), jnp.int32) for _ in range(6)]
          + [pltpu.VMEM((CHA,), jnp.int32) for _ in range(6)]
          + [pltpu.VMEM_SHARED((N, SLAB), jnp.float32)]
          + [pltpu.SemaphoreType.DMA for _ in range(12)]
      ),
  )
  def k(h_hbm, src_hbm, dst_hbm, z_hbm, out_hbm, *scratch):
    rows_b = scratch[0:3]
    srcc_b = scratch[3:9]
    dstc_b = scratch[9:15]
    agg_sh = scratch[15]
    semg = scratch[16:19]
    semi = scratch[19:25]
    sems = scratch[25:28]
    rows_v = rows_b[0]
    cid = lax.axis_index("c")
    sid = lax.axis_index("s")
    base = pl.multiple_of(sid * RB, 8)

    def zero_acc():
      for j in range(RB // ZC):
        pltpu.async_copy(rows_v, agg_sh.at[pl.ds(base + j * ZC, ZC)], sems[0])
      rem = RB - (RB // ZC) * ZC
      if rem:
        pltpu.async_copy(rows_v.at[pl.ds(0, rem)],
                         agg_sh.at[pl.ds(base + RB - rem, rem)], sems[1])

      @pl.when(sid == 0)
      def _():
        pltpu.async_copy(rows_v.at[pl.ds(0, N - RB * NSUB)],
                         agg_sh.at[pl.ds(RB * NSUB, N - RB * NSUB)], sems[2])

      for j in range(RB // ZC):
        pltpu.make_async_copy(rows_v, agg_sh.at[pl.ds(base + j * ZC, ZC)],
                              sems[0]).wait()
      if rem:
        pltpu.make_async_copy(rows_v.at[pl.ds(0, rem)],
                              agg_sh.at[pl.ds(base + RB - rem, rem)],
                              sems[1]).wait()

      @pl.when(sid == 0)
      def _():
        pltpu.make_async_copy(rows_v.at[pl.ds(0, N - RB * NSUB)],
                              agg_sh.at[pl.ds(RB * NSUB, N - RB * NSUB)],
                              sems[2]).wait()

    def edge_loop(ch, nch, src_base, dst_base):
      if ch == CHA:
        rows = list(rows_b)
        srcc = list(srcc_b)
        dstc = list(dstc_b)
      else:
        rows = [r.at[pl.ds(0, ch)] for r in rows_b]
        srcc = [r.at[pl.ds(0, ch)] for r in srcc_b]
        dstc = [r.at[pl.ds(0, ch)] for r in dstc_b]

      def idx_issue(kk, r):
        off = pl.multiple_of(kk * ch, 8)
        pltpu.async_copy(src_hbm.at[pl.ds(src_base + off, ch)],
                         srcc[r], semi[r])
        pltpu.async_copy(dst_hbm.at[pl.ds(dst_base + off, ch)],
                         dstc[r], semi[r])

      def idx_wait(kk, r):
        off = pl.multiple_of(kk * ch, 8)
        pltpu.make_async_copy(src_hbm.at[pl.ds(src_base + off, ch)],
                              srcc[r], semi[r]).wait()
        pltpu.make_async_copy(dst_hbm.at[pl.ds(dst_base + off, ch)],
                              dstc[r], semi[r]).wait()

      def gather_issue(r3, r6):
        pltpu.async_copy(h_hbm.at[srcc[r6]], rows[r3], semg[r3])

      def gather_wait(r3, r6):
        pltpu.make_async_copy(h_hbm.at[srcc[r6]], rows[r3], semg[r3]).wait()

      def scatter_issue(r3, r6):
        pltpu.async_copy(rows[r3], agg_sh.at[dstc[r6]], sems[r3], add=True)

      def scatter_wait(r3, r6):
        pltpu.make_async_copy(rows[r3], agg_sh.at[dstc[r6]],
                              sems[r3]).wait()

      for r in range(3):
        idx_issue(r, r)
      idx_wait(0, 0)
      gather_issue(0, 0)
      idx_wait(1, 1)
      gather_issue(1, 1)

      def body(p, carry):
        k0 = 6 * p
        for r in range(6):
          kk = k0 + r

          @pl.when(kk < nch)
          def _(kk=kk, r=r):
            gather_wait(r % 3, r)
            scatter_issue(r % 3, r)

            @pl.when(kk >= 1)
            def _():
              scatter_wait((r - 1) % 3, (r - 1) % 6)

            @pl.when(kk + 3 < nch)
            def _():
              idx_issue(kk + 3, (r + 3) % 6)

            @pl.when(kk + 2 < nch)
            def _():
              idx_wait(kk + 2, (r + 2) % 6)
              gather_issue((r + 2) % 3, (r + 2) % 6)

        return carry

      lax.fori_loop(0, (nch + 5) // 6, body, 0)
      scatter_wait((nch - 1) % 3, (nch - 1) % 6)

    def writeback(slot):
      for j in range(RB // ZC):
        pltpu.async_copy(agg_sh.at[pl.ds(base + j * ZC, ZC)],
                         out_hbm.at[slot, pl.ds(base + j * ZC, ZC)], semg[0])
      rem = RB - (RB // ZC) * ZC
      if rem:
        pltpu.async_copy(agg_sh.at[pl.ds(base + RB - rem, rem)],
                         out_hbm.at[slot, pl.ds(base + RB - rem, rem)],
                         semg[1])

      @pl.when(sid == 0)
      def _():
        tail = N - RB * NSUB
        pltpu.async_copy(agg_sh.at[pl.ds(RB * NSUB, tail)],
                         out_hbm.at[slot, pl.ds(RB * NSUB, tail)], semg[2])

      for j in range(RB // ZC):
        pltpu.make_async_copy(agg_sh.at[pl.ds(base + j * ZC, ZC)],
                              out_hbm.at[slot, pl.ds(base + j * ZC, ZC)],
                              semg[0]).wait()
      if rem:
        pltpu.make_async_copy(agg_sh.at[pl.ds(base + RB - rem, rem)],
                              out_hbm.at[slot, pl.ds(base + RB - rem, rem)],
                              semg[1]).wait()

      @pl.when(sid == 0)
      def _():
        tail = N - RB * NSUB
        pltpu.make_async_copy(agg_sh.at[pl.ds(RB * NSUB, tail)],
                              out_hbm.at[slot, pl.ds(RB * NSUB, tail)],
                              semg[2]).wait()

    with jax.named_scope("zeroA"):
      pltpu.sync_copy(z_hbm, rows_v)
      zero_acc()
      plsc.subcore_barrier()
    with jax.named_scope("edgesA"):
      edge_loop(CHA, NCHA, cid * E + sid * EPT, sid * EPT)
      plsc.subcore_barrier()
    with jax.named_scope("wbA"):
      writeback(cid)

    with jax.named_scope("zeroB"):
      pltpu.sync_copy(z_hbm, rows_v)
      zero_acc()
      plsc.subcore_barrier()
    with jax.named_scope("edgesB"):
      edge_loop(CHB, NCHB, 2 * E + (cid * NSUB + sid) * EPT2,
                (cid * NSUB + sid) * EPT2)
      plsc.subcore_barrier()
    with jax.named_scope("wbB"):
      writeback(2 + cid)

  return k(h_slabs, src_all, dst_all, zeros_blk)


def _split_store(ref, val):
  for s in range(NSLAB):
    ref[s] = val[:, s * SLAB:(s + 1) * SLAB]


def _tc_lift(nf, Wl, bl, W2, b2):
  def body(nf_ref, Wl_ref, bl_ref, W2_ref, b2_ref, x_ref, h_ref):
    x0 = _dot(nf_ref[...], Wl_ref[...]) + bl_ref[...]
    h1 = jnp.maximum(_dot(x0, W2_ref[...]) + b2_ref[...], 0.0)
    _split_store(x_ref, x0)
    _split_store(h_ref, h1)

  return pl.pallas_call(
      body,
      grid=(N // NB,),
      in_specs=[
          pl.BlockSpec((NB, DP), lambda i: (i, 0)),
          pl.BlockSpec((DP, HP), lambda i: (0, 0)),
          pl.BlockSpec((1, HP), lambda i: (0, 0)),
          pl.BlockSpec((HP, HP), lambda i: (0, 0)),
          pl.BlockSpec((1, HP), lambda i: (0, 0)),
      ],
      out_specs=[
          pl.BlockSpec((NSLAB, NB, SLAB), lambda i: (0, i, 0)),
          pl.BlockSpec((NSLAB, NB, SLAB), lambda i: (0, i, 0)),
      ],
      out_shape=[
          jax.ShapeDtypeStruct((NSLAB, N, SLAB), jnp.float32),
          jax.ShapeDtypeStruct((NSLAB, N, SLAB), jnp.float32),
      ],
  )(nf, Wl, bl, W2, b2)


def _select_updated(agg_ref, xp_ref):
  s2 = agg_ref[2] + agg_ref[3]
  has = s2[:, DEGC:DEGC + 1] > 0.0
  return jnp.concatenate(
      [jnp.where(has, agg_ref[0], xp_ref[0]),
       jnp.where(has, agg_ref[1], xp_ref[1]),
       jnp.where(has, s2, xp_ref[2])], axis=1)


def _tc_layer(agg, xp, W1, b1, W2n, b2n):
  def body(agg_ref, xp_ref, W1_ref, b1_ref, W2_ref, b2_ref, x_ref, h_ref):
    xm = _select_updated(agg_ref, xp_ref)
    xn = jnp.maximum(_dot(xm, W1_ref[...]) + b1_ref[...], 0.0)
    h = jnp.maximum(_dot(xn, W2_ref[...]) + b2_ref[...], 0.0)
    _split_store(x_ref, xn)
    _split_store(h_ref, h)

  return pl.pallas_call(
      body,
      grid=(N // NB,),
      in_specs=[
          pl.BlockSpec((4, NB, SLAB), lambda i: (0, i, 0)),
          pl.BlockSpec((NSLAB, NB, SLAB), lambda i: (0, i, 0)),
          pl.BlockSpec((HP, HP), lambda i: (0, 0)),
          pl.BlockSpec((1, HP), lambda i: (0, 0)),
          pl.BlockSpec((HP, HP), lambda i: (0, 0)),
          pl.BlockSpec((1, HP), lambda i: (0, 0)),
      ],
      out_specs=[
          pl.BlockSpec((NSLAB, NB, SLAB), lambda i: (0, i, 0)),
          pl.BlockSpec((NSLAB, NB, SLAB), lambda i: (0, i, 0)),
      ],
      out_shape=[
          jax.ShapeDtypeStruct((NSLAB, N, SLAB), jnp.float32),
          jax.ShapeDtypeStruct((NSLAB, N, SLAB), jnp.float32),
      ],
  )(agg, xp, W1, b1, W2n, b2n)


def _tc_final(agg, xp, W1, b1, Wr, br, onehot):
  def body(agg_ref, xp_ref, W1_ref, b1_ref, Wr_ref, br_ref, oh_ref, out_ref):
    i = pl.program_id(0)
    xm = _select_updated(agg_ref, xp_ref)
    x3 = jnp.maximum(_dot(xm, W1_ref[...]) + b1_ref[...], 0.0)
    nl = _dot(x3, Wr_ref[...]) + br_ref[...]
    contrib = _dot_hi(oh_ref[0], nl)

    @pl.when(i == 0)
    def _():
      out_ref[...] = jnp.zeros_like(out_ref)

    out_ref[...] += contrib

  return pl.pallas_call(
      body,
      grid=(N // NB,),
      in_specs=[
          pl.BlockSpec((4, NB, SLAB), lambda i: (0, i, 0)),
          pl.BlockSpec((NSLAB, NB, SLAB), lambda i: (0, i, 0)),
          pl.BlockSpec((HP, HP), lambda i: (0, 0)),
          pl.BlockSpec((1, HP), lambda i: (0, 0)),
          pl.BlockSpec((HP, CPAD), lambda i: (0, 0)),
          pl.BlockSpec((1, CPAD), lambda i: (0, 0)),
          pl.BlockSpec((1, GP, NB), lambda i: (i, 0, 0)),
      ],
      out_specs=pl.BlockSpec((GP, CPAD), lambda i: (0, 0)),
      out_shape=jax.ShapeDtypeStruct((GP, CPAD), jnp.float32),
  )(agg, xp, W1, b1, Wr, br, onehot)


def kernel(node_feats, edge_index, graph_ids, W_lift, b_lift,
           W2_1, b2_1, W1_1, b1_1,
           W2_2, b2_2, W1_2, b1_2,
           W2_3, b2_3, W1_3, b1_3,
           W_read, b_read):
  f32 = jnp.float32
  h_dim = W2_1.shape[0]
  nf_p = jnp.pad(node_feats, ((0, 0), (0, DP - node_feats.shape[1])))
  Wl = jnp.pad(W_lift, ((0, DP - W_lift.shape[0]), (0, HP - W_lift.shape[1])))
  bl = jnp.pad(b_lift, (0, HP - b_lift.shape[0]))[None, :]

  def pad_w(w):
    return jnp.pad(w, ((0, HP - w.shape[0]), (0, HP - w.shape[1])))

  def pad_b(b, deg_one=False):
    bp = jnp.pad(b, (0, HP - b.shape[0]))
    if deg_one:
      bp = bp.at[h_dim].set(1.0)
    return bp[None, :]

  W2s = (pad_w(W2_1), pad_w(W2_2), pad_w(W2_3))
  b2s = (pad_b(b2_1, True), pad_b(b2_2, True), pad_b(b2_3, True))
  W1s = (pad_w(W1_1), pad_w(W1_2), pad_w(W1_3))
  b1s = (pad_b(b1_1), pad_b(b1_2), pad_b(b1_3))
  Wr = jnp.pad(W_read,
               ((0, HP - W_read.shape[0]), (0, CPAD - W_read.shape[1])))
  br = jnp.pad(b_read, (0, CPAD - b_read.shape[0]))[None, :]

  src = edge_index[0]
  dst = edge_index[1]
  src_all = jnp.reshape(
      jnp.stack([src, src + N, src + 2 * N]), (NSLAB * E,))
  zeros_blk = jnp.zeros((ZC, SLAB), f32)
  onehot = (graph_ids[None, :]
            == jnp.arange(GP, dtype=jnp.int32)[:, None]).astype(f32)
  onehot = jnp.transpose(jnp.reshape(onehot, (GP, N // NB, NB)), (1, 0, 2))

  x_st, h_st = _tc_lift(nf_p, Wl, bl, W2s[0], b2s[0])
  out = None
  for i in range(3):
    agg = _sc_edge_pass(jnp.reshape(h_st, (NSLAB * N, SLAB)),
                        src_all, dst, zeros_blk)
    if i < 2:
      x_st, h_st = _tc_layer(agg, x_st, W1s[i], b1s[i], W2s[i + 1], b2s[i + 1])
    else:
      out = _tc_final(agg, x_st, W1s[2], b1s[2], Wr, br, onehot)
  return out[:10, :W_read.shape[1]]
